# baseline (speedup 1.0000x reference)
import jax
import jax.numpy as jnp
from jax import lax
from jax.experimental import pallas as pl
from jax.experimental.pallas import tpu as pltpu
from jax.experimental.pallas import tpu_sc as plsc

NC = 2
NS = 16
NW = NC * NS
ED = 160000
NSEG = 5000
DM = 256
HC = 512

RB = 160
CAP = 5504
STG = 3200
DC = 32


def _worker(cid, sid):
    return sid * NC + cid


def _segsum(src, dst, x):
    n = NSEG
    x = jnp.concatenate([x, jnp.zeros((8, DM), jnp.float32)], axis=0)
    nstg = ED // STG
    nin = STG // 16
    big = jnp.int32(RB)
    sent = jnp.int32(n)

    mesh = plsc.VectorSubcoreMesh(core_axis_name="c", subcore_axis_name="s")
    out_type = (jax.ShapeDtypeStruct((NW * RB, DM), jnp.float32),
                jax.ShapeDtypeStruct((NW * RB, 16), jnp.float32))
    scratch = (
        pltpu.VMEM((RB, DM), jnp.float32),
        pltpu.VMEM((RB, 16), jnp.float32),
        pltpu.VMEM((2 * STG,), jnp.int32),
        pltpu.VMEM((CAP + 16 + 2 * DC,), jnp.int32),
        pltpu.VMEM((DC, DM), jnp.float32),
        pltpu.VMEM((DC, DM), jnp.float32),
        pltpu.SemaphoreType.DMA,
        pltpu.SemaphoreType.DMA,
    )

    def body(src_hbm, dst_hbm, x_hbm, s_hbm, c_hbm, acc, cnt, stg, cpk,
             rows0, rows1, sem0, sem1):
        cid = lax.axis_index("c")
        sid = lax.axis_index("s")
        lo = _worker(cid, sid) * RB
        zf16 = jnp.zeros((16,), jnp.float32)
        one0 = jnp.where(lax.iota(jnp.int32, 16) == 0, 1.0, 0.0)

        def za(r, _):
            def zc(c, _):
                acc[r, pl.ds(c * 16, 16)] = zf16
                return 0
            lax.fori_loop(0, DM // 16, zc, 0)
            cnt[r, pl.ds(0, 16)] = zf16
            return 0
        lax.fori_loop(0, RB, za, 0)

        def outer(jo, cur):
            off = jo * STG
            pltpu.sync_copy(dst_hbm.at[pl.ds(off, STG)], stg.at[pl.ds(0, STG)])
            pltpu.sync_copy(src_hbm.at[pl.ds(off, STG)],
                            stg.at[pl.ds(STG, STG)])

            def inner(k, cur):
                dv = stg[pl.ds(k * 16, 16)]
                sv = stg[pl.ds(STG + k * 16, 16)]
                dl = dv - lo
                m = (dl >= 0) & (dl < RB)
                np16 = plsc.all_reduce_population_count(m)
                npop = np16[0]

                @pl.when(npop > 0)
                def _():
                    key = jnp.where(m, dl, big)
                    sk, sv2 = plsc.sort_key_val(key, sv)
                    curc = jnp.minimum(cur, CAP)
                    cpk[pl.ds(curc, 16)] = (sk << 13) | sv2
                return cur + npop
            return lax.fori_loop(0, nin, inner, cur)
        cur = lax.fori_loop(0, nstg, outer, 0)
        cur = jnp.minimum(cur, CAP)

        for t in range(DC // 16):
            cpk[pl.ds(cur + t * 16, 16)] = jnp.full((16,), sent, jnp.int32)

        nch = (cur + DC - 1) // DC
        gbase = CAP + 16
        prow = (rows0, rows1)
        psem = (sem0, sem1)

        def prep(j, p):
            gb = gbase + p * DC
            for g in range(DC // 16):
                pv = cpk[pl.ds(j * DC + g * 16, 16)]
                cpk[pl.ds(gb + g * 16, 16)] = pv & 8191
            pltpu.async_copy(x_hbm.at[cpk.at[pl.ds(gb, DC)]],
                             prow[p], psem[p])

        def rmw(j, p):
            gb = gbase + p * DC
            pltpu.make_async_copy(x_hbm.at[cpk.at[pl.ds(gb, DC)]],
                                  prow[p], psem[p]).wait()
            for g in range(DC // 16):
                iv = cpk[pl.ds(j * DC + g * 16, 16)] >> 13
                for lane in range(16):
                    dl = iv[lane]
                    r = g * 16 + lane
                    for c in range(DM // 16):
                        acc[dl, pl.ds(c * 16, 16)] = (
                            acc[dl, pl.ds(c * 16, 16)]
                            + prow[p][r, pl.ds(c * 16, 16)])
                    cnt[dl, pl.ds(0, 16)] = cnt[dl, pl.ds(0, 16)] + one0

        @pl.when(nch > 0)
        def _():
            prep(0, 0)

        def pairstep(i2, _):
            j0 = 2 * i2

            @pl.when(j0 < nch)
            def _():
                @pl.when(j0 + 1 < nch)
                def _():
                    prep(j0 + 1, 1)
                rmw(j0, 0)

            @pl.when(j0 + 1 < nch)
            def _():
                @pl.when(j0 + 2 < nch)
                def _():
                    prep(j0 + 2, 0)
                rmw(j0 + 1, 1)
            return 0
        lax.fori_loop(0, (nch + 1) // 2, pairstep, 0)
        nsent = (nch * DC - cur).astype(jnp.float32)
        cnt[0, pl.ds(0, 16)] = cnt[0, pl.ds(0, 16)] - nsent * one0

        pltpu.sync_copy(acc.at[pl.ds(0, RB)], s_hbm.at[pl.ds(lo, RB)])
        pltpu.sync_copy(cnt.at[pl.ds(0, RB)], c_hbm.at[pl.ds(lo, RB)])

    kern = pl.kernel(
        body, out_type=out_type, mesh=mesh, scratch_types=scratch,
        compiler_params=pltpu.CompilerParams(needs_layout_passes=False))
    s, c = kern(src, dst, x)
    return s[:n], c[:n]


def _gather_pair(a, b, row, col):
    d = a.shape[1]
    ept = ED // NW
    C = 32
    nfull = ept // C
    rem = ept - nfull * C
    mesh = plsc.VectorSubcoreMesh(core_axis_name="c", subcore_axis_name="s")
    out_type = (jax.ShapeDtypeStruct((ED, d), jnp.float32),
                jax.ShapeDtypeStruct((ED, d), jnp.float32))
    scratch = (
        pltpu.VMEM((C,), jnp.int32), pltpu.VMEM((C,), jnp.int32),
        pltpu.VMEM((C,), jnp.int32), pltpu.VMEM((C,), jnp.int32),
        pltpu.VMEM((C, d), jnp.float32), pltpu.VMEM((C, d), jnp.float32),
        pltpu.VMEM((C, d), jnp.float32), pltpu.VMEM((C, d), jnp.float32),
        pltpu.VMEM((rem,), jnp.int32), pltpu.VMEM((rem,), jnp.int32),
        pltpu.VMEM((rem, d), jnp.float32), pltpu.VMEM((rem, d), jnp.float32),
        pltpu.SemaphoreType.DMA, pltpu.SemaphoreType.DMA,
        pltpu.SemaphoreType.DMA, pltpu.SemaphoreType.DMA,
    )

    def body(a_hbm, b_hbm, row_hbm, col_hbm, ga_hbm, gb_hbm,
             ri0, ci0, ri1, ci1, ba0, bb0, ba1, bb1,
             ri8, ci8, ba8, bb8, semg, semg2, semo, semo2):
        cid = lax.axis_index("c")
        sid = lax.axis_index("s")
        base = _worker(cid, sid) * ept
        ribufs = (ri0, ri1)
        cibufs = (ci0, ci1)
        babufs = (ba0, ba1)
        bbbufs = (bb0, bb1)
        gsems = (semg, semg2)
        osems = (semo, semo2)

        def fetch(j, p):
            off = base + j * C
            pltpu.sync_copy(row_hbm.at[pl.ds(off, C)], ribufs[p])
            pltpu.sync_copy(col_hbm.at[pl.ds(off, C)], cibufs[p])
            ca = pltpu.async_copy(a_hbm.at[ribufs[p]], babufs[p], gsems[p])
            cb = pltpu.async_copy(b_hbm.at[cibufs[p]], bbbufs[p], gsems[p])
            return ca, cb

        def flush(j, p):
            off = base + j * C
            pltpu.async_copy(babufs[p], ga_hbm.at[pl.ds(off, C)], osems[p])
            pltpu.async_copy(bbbufs[p], gb_hbm.at[pl.ds(off, C)], osems[p])

        def wait_gather(p):
            pltpu.make_async_copy(a_hbm.at[ribufs[p]], babufs[p],
                                  gsems[p]).wait()
            pltpu.make_async_copy(b_hbm.at[cibufs[p]], bbbufs[p],
                                  gsems[p]).wait()

        def wait_out(p):
            pltpu.make_async_copy(babufs[p], ga_hbm.at[pl.ds(0, C)],
                                  osems[p]).wait()
            pltpu.make_async_copy(bbbufs[p], gb_hbm.at[pl.ds(0, C)],
                                  osems[p]).wait()

        fetch(0, 0)

        def sub(j, p, q):
            @pl.when(j + 1 < nfull)
            def _():
                @pl.when(j >= 1)
                def _():
                    wait_out(q)
                fetch(j + 1, q)
            wait_gather(p)
            flush(j, p)

        def pairstep(i2, _):
            sub(i2 * 2, 0, 1)
            sub(i2 * 2 + 1, 1, 0)
            return 0
        lax.fori_loop(0, nfull // 2, pairstep, 0)
        wait_out(0)
        wait_out(1)

        offr = base + nfull * C
        pltpu.sync_copy(row_hbm.at[pl.ds(offr, rem)], ri8)
        pltpu.sync_copy(col_hbm.at[pl.ds(offr, rem)], ci8)
        ca = pltpu.async_copy(a_hbm.at[ri8], ba8, semg)
        cb = pltpu.async_copy(b_hbm.at[ci8], bb8, semg)
        ca.wait()
        cb.wait()
        pltpu.sync_copy(ba8, ga_hbm.at[pl.ds(offr, rem)])
        pltpu.sync_copy(bb8, gb_hbm.at[pl.ds(offr, rem)])

    kern = pl.kernel(body, out_type=out_type, mesh=mesh, scratch_types=scratch)
    return kern(a, b, row, col)


def _dot(a, b):
    return jnp.dot(a, b, preferred_element_type=jnp.float32)


def _sage_linear(s, c, x_dst, Wl, bl, Wr, relu, Wproj=None, bproj=None):
    n, d = x_dst.shape
    R = 1000
    dproj = None if Wproj is None else Wproj.shape[1]

    def body(s_ref, c_ref, x_ref, wl_ref, b_ref, wr_ref, *rest):
        o_ref = rest[-1]
        cnt = c_ref[...][:, 0]
        mean = s_ref[...] / jnp.maximum(cnt, 1.0)[:, None]
        z = _dot(mean, wl_ref[...]) + b_ref[...] + _dot(x_ref[...], wr_ref[...])
        if relu:
            z = jnp.maximum(z, 0.0)
        if Wproj is not None:
            z = _dot(z, rest[0][...]) + rest[1][...]
        o_ref[...] = z

    in_specs = [
        pl.BlockSpec((R, d), lambda i: (i, 0)),
        pl.BlockSpec((R, 16), lambda i: (i, 0)),
        pl.BlockSpec((R, d), lambda i: (i, 0)),
        pl.BlockSpec((d, d), lambda i: (0, 0)),
        pl.BlockSpec((1, d), lambda i: (0, 0)),
        pl.BlockSpec((d, d), lambda i: (0, 0)),
    ]
    args = [s, c, x_dst, Wl, bl.reshape(1, d), Wr]
    dout = d
    if Wproj is not None:
        in_specs += [pl.BlockSpec((d, dproj), lambda i: (0, 0)),
                     pl.BlockSpec((1, dproj), lambda i: (0, 0))]
        args += [Wproj, bproj.reshape(1, dproj)]
        dout = dproj
    return pl.pallas_call(
        body,
        grid=(n // R,),
        in_specs=in_specs,
        out_specs=pl.BlockSpec((R, dout), lambda i: (i, 0)),
        out_shape=jax.ShapeDtypeStruct((n, dout), jnp.float32),
    )(*args)


def _decoder_mlp(ga, gb, W2, b2, W3, b3, w4row, b4):
    R = 640

    def body(ga_ref, gb_ref, w2_ref, b2_ref, w3_ref, b3_ref, w4_ref, b4_ref,
             o_ref):
        z = jnp.maximum(ga_ref[...] + gb_ref[...], 0.0)
        z = jnp.maximum(_dot(z, w2_ref[...]) + b2_ref[...], 0.0)
        z = jnp.maximum(_dot(z, w3_ref[...]) + b3_ref[...], 0.0)
        logit = jnp.sum(z * w4_ref[...], axis=1) + b4_ref[0]
        o_ref[...] = (jax.nn.sigmoid(logit) * 4.0 + 1.0)[:, None]

    return pl.pallas_call(
        body,
        grid=(ED // R,),
        in_specs=[
            pl.BlockSpec((R, HC), lambda i: (i, 0)),
            pl.BlockSpec((R, HC), lambda i: (i, 0)),
            pl.BlockSpec((HC, HC), lambda i: (0, 0)),
            pl.BlockSpec((1, HC), lambda i: (0, 0)),
            pl.BlockSpec((HC, HC), lambda i: (0, 0)),
            pl.BlockSpec((1, HC), lambda i: (0, 0)),
            pl.BlockSpec((1, HC), lambda i: (0, 0)),
            pl.BlockSpec(memory_space=pltpu.SMEM),
        ],
        out_specs=pl.BlockSpec((R, 1), lambda i: (i, 0)),
        out_shape=jax.ShapeDtypeStruct((ED, 1), jnp.float32),
    )(ga, gb, W2, b2.reshape(1, HC), W3, b3.reshape(1, HC), w4row,
      b4).reshape(ED)


def kernel(x_user, x_item, edge_index_ui, edge_index_iu, edge_label_index,
           Wl_ui1, bl_ui1, Wr_ui1, Wl_iu1, bl_iu1, Wr_iu1,
           Wl_ui2, bl_ui2, Wr_ui2, Wl_iu2, bl_iu2, Wr_iu2,
           W1, b1, W2, b2, W3, b3, W4, b4):
    src_ui, dst_ui = edge_index_ui[0], edge_index_ui[1]
    src_iu, dst_iu = edge_index_iu[0], edge_index_iu[1]
    row, col = edge_label_index[0], edge_label_index[1]

    Sui, Cui = _segsum(src_ui, dst_ui, x_user)
    Siu, Ciu = _segsum(src_iu, dst_iu, x_item)
    h_item = _sage_linear(Sui, Cui, x_item, Wl_ui1, bl_ui1, Wr_ui1, relu=True)
    h_user = _sage_linear(Siu, Ciu, x_user, Wl_iu1, bl_iu1, Wr_iu1, relu=True)

    S2ui, _ = _segsum(src_ui, dst_ui, h_user)
    S2iu, _ = _segsum(src_iu, dst_iu, h_item)
    a = _sage_linear(S2iu, Ciu, h_user, Wl_iu2, bl_iu2, Wr_iu2, relu=False,
                     Wproj=W1[:DM], bproj=jnp.zeros((HC,), jnp.float32))
    b = _sage_linear(S2ui, Cui, h_item, Wl_ui2, bl_ui2, Wr_ui2, relu=False,
                     Wproj=W1[DM:], bproj=b1)

    ga, gb = _gather_pair(a, b, row, col)
    out = _decoder_mlp(ga, gb, W2, b2, W3, b3, W4.reshape(1, HC), b4)
    mask = jnp.ones((edge_label_index.shape[1],), dtype=bool)
    return (out, mask)


# speedup vs baseline: 1.0658x; 1.0658x over previous
import jax
import jax.numpy as jnp
from jax import lax
from jax.experimental import pallas as pl
from jax.experimental.pallas import tpu as pltpu
from jax.experimental.pallas import tpu_sc as plsc

NC = 2
NS = 16
NW = NC * NS
ED = 160000
NSEG = 5000
DM = 256
HC = 512

RB = 160
CAP = 5504
STG = 3200
DC = 32


def _worker(cid, sid):
    return sid * NC + cid


def _segsum(src, dst, x):
    n = NSEG
    x = jnp.concatenate([x, jnp.zeros((8, DM), jnp.float32)], axis=0)
    nstg = ED // STG
    nin = STG // 16
    big = jnp.int32(RB)
    sent = jnp.int32(n)

    mesh = plsc.VectorSubcoreMesh(core_axis_name="c", subcore_axis_name="s")
    out_type = (jax.ShapeDtypeStruct((NW * RB, DM), jnp.float32),
                jax.ShapeDtypeStruct((NW * RB, 16), jnp.float32))
    scratch = (
        pltpu.VMEM((RB, DM), jnp.float32),
        pltpu.VMEM((RB, 16), jnp.float32),
        pltpu.VMEM((2 * STG,), jnp.int32),
        pltpu.VMEM((2 * STG,), jnp.int32),
        pltpu.VMEM((CAP + 16 + 2 * DC,), jnp.int32),
        pltpu.VMEM((DC, DM), jnp.float32),
        pltpu.VMEM((DC, DM), jnp.float32),
        pltpu.SemaphoreType.DMA,
        pltpu.SemaphoreType.DMA,
        pltpu.SemaphoreType.DMA,
        pltpu.SemaphoreType.DMA,
    )

    def body(src_hbm, dst_hbm, x_hbm, s_hbm, c_hbm, acc, cnt, stg0, stg1,
             cpk, rows0, rows1, sem0, sem1, ssem0, ssem1):
        cid = lax.axis_index("c")
        sid = lax.axis_index("s")
        lo = _worker(cid, sid) * RB
        zf16 = jnp.zeros((16,), jnp.float32)
        one0 = jnp.where(lax.iota(jnp.int32, 16) == 0, 1.0, 0.0)

        def za(r, _):
            def zc(c, _):
                acc[r, pl.ds(c * 16, 16)] = zf16
                return 0
            lax.fori_loop(0, DM // 16, zc, 0)
            cnt[r, pl.ds(0, 16)] = zf16
            return 0
        lax.fori_loop(0, RB, za, 0)

        pstg = (stg0, stg1)
        pssem = (ssem0, ssem1)

        def sfetch(jo, p):
            off = jo * STG
            pltpu.async_copy(dst_hbm.at[pl.ds(off, STG)],
                             pstg[p].at[pl.ds(0, STG)], pssem[p])
            pltpu.async_copy(src_hbm.at[pl.ds(off, STG)],
                             pstg[p].at[pl.ds(STG, STG)], pssem[p])

        def swait(jo, p):
            off = jo * STG
            pltpu.make_async_copy(dst_hbm.at[pl.ds(off, STG)],
                                  pstg[p].at[pl.ds(0, STG)], pssem[p]).wait()
            pltpu.make_async_copy(src_hbm.at[pl.ds(off, STG)],
                                  pstg[p].at[pl.ds(STG, STG)], pssem[p]).wait()

        def sblock(jo, p, cur):
            swait(jo, p)
            stg = pstg[p]

            def inner(k, cur):
                dv = stg[pl.ds(k * 16, 16)]
                sv = stg[pl.ds(STG + k * 16, 16)]
                dl = dv - lo
                m = (dl >= 0) & (dl < RB)
                np16 = plsc.all_reduce_population_count(m)
                npop = np16[0]

                @pl.when(npop > 0)
                def _():
                    key = jnp.where(m, dl, big)
                    sk, sv2 = plsc.sort_key_val(key, sv)
                    curc = jnp.minimum(cur, CAP)
                    cpk[pl.ds(curc, 16)] = (sk << 13) | sv2
                return cur + npop
            return lax.fori_loop(0, nin, inner, cur, unroll=4)

        sfetch(0, 0)

        def souter(i2, cur):
            jo = 2 * i2

            @pl.when(jo + 1 < nstg)
            def _():
                sfetch(jo + 1, 1)
            cur = sblock(jo, 0, cur)

            @pl.when(jo + 2 < nstg)
            def _():
                sfetch(jo + 2, 0)
            return lax.cond(jo + 1 < nstg,
                            lambda c: sblock(jo + 1, 1, c),
                            lambda c: c, cur)
        cur = lax.fori_loop(0, (nstg + 1) // 2, souter, 0)
        cur = jnp.minimum(cur, CAP)

        for t in range(DC // 16):
            cpk[pl.ds(cur + t * 16, 16)] = jnp.full((16,), sent, jnp.int32)

        nch = (cur + DC - 1) // DC
        gbase = CAP + 16
        prow = (rows0, rows1)
        psem = (sem0, sem1)

        def prep(j, p):
            gb = gbase + p * DC
            for g in range(DC // 16):
                pv = cpk[pl.ds(j * DC + g * 16, 16)]
                cpk[pl.ds(gb + g * 16, 16)] = pv & 8191
            pltpu.async_copy(x_hbm.at[cpk.at[pl.ds(gb, DC)]],
                             prow[p], psem[p])

        def rmw(j, p):
            gb = gbase + p * DC
            pltpu.make_async_copy(x_hbm.at[cpk.at[pl.ds(gb, DC)]],
                                  prow[p], psem[p]).wait()
            for g in range(DC // 16):
                iv = cpk[pl.ds(j * DC + g * 16, 16)] >> 13
                for lane in range(16):
                    dl = iv[lane]
                    r = g * 16 + lane
                    for c in range(DM // 16):
                        acc[dl, pl.ds(c * 16, 16)] = (
                            acc[dl, pl.ds(c * 16, 16)]
                            + prow[p][r, pl.ds(c * 16, 16)])
                    cnt[dl, pl.ds(0, 16)] = cnt[dl, pl.ds(0, 16)] + one0

        @pl.when(nch > 0)
        def _():
            prep(0, 0)

        def pairstep(i2, _):
            j0 = 2 * i2

            @pl.when(j0 < nch)
            def _():
                @pl.when(j0 + 1 < nch)
                def _():
                    prep(j0 + 1, 1)
                rmw(j0, 0)

            @pl.when(j0 + 1 < nch)
            def _():
                @pl.when(j0 + 2 < nch)
                def _():
                    prep(j0 + 2, 0)
                rmw(j0 + 1, 1)
            return 0
        lax.fori_loop(0, (nch + 1) // 2, pairstep, 0)
        nsent = (nch * DC - cur).astype(jnp.float32)
        cnt[0, pl.ds(0, 16)] = cnt[0, pl.ds(0, 16)] - nsent * one0

        pltpu.sync_copy(acc.at[pl.ds(0, RB)], s_hbm.at[pl.ds(lo, RB)])
        pltpu.sync_copy(cnt.at[pl.ds(0, RB)], c_hbm.at[pl.ds(lo, RB)])

    kern = pl.kernel(
        body, out_type=out_type, mesh=mesh, scratch_types=scratch,
        compiler_params=pltpu.CompilerParams(needs_layout_passes=False))
    s, c = kern(src, dst, x)
    return s[:n], c[:n]


def _gather_pair(a, b, row, col):
    d = a.shape[1]
    ept = ED // NW
    C = 32
    nfull = ept // C
    rem = ept - nfull * C
    mesh = plsc.VectorSubcoreMesh(core_axis_name="c", subcore_axis_name="s")
    out_type = (jax.ShapeDtypeStruct((ED, d), jnp.float32),
                jax.ShapeDtypeStruct((ED, d), jnp.float32))
    scratch = (
        pltpu.VMEM((C,), jnp.int32), pltpu.VMEM((C,), jnp.int32),
        pltpu.VMEM((C,), jnp.int32), pltpu.VMEM((C,), jnp.int32),
        pltpu.VMEM((C, d), jnp.float32), pltpu.VMEM((C, d), jnp.float32),
        pltpu.VMEM((C, d), jnp.float32), pltpu.VMEM((C, d), jnp.float32),
        pltpu.VMEM((rem,), jnp.int32), pltpu.VMEM((rem,), jnp.int32),
        pltpu.VMEM((rem, d), jnp.float32), pltpu.VMEM((rem, d), jnp.float32),
        pltpu.SemaphoreType.DMA, pltpu.SemaphoreType.DMA,
        pltpu.SemaphoreType.DMA, pltpu.SemaphoreType.DMA,
    )

    def body(a_hbm, b_hbm, row_hbm, col_hbm, ga_hbm, gb_hbm,
             ri0, ci0, ri1, ci1, ba0, bb0, ba1, bb1,
             ri8, ci8, ba8, bb8, semg, semg2, semo, semo2):
        cid = lax.axis_index("c")
        sid = lax.axis_index("s")
        base = _worker(cid, sid) * ept
        ribufs = (ri0, ri1)
        cibufs = (ci0, ci1)
        babufs = (ba0, ba1)
        bbbufs = (bb0, bb1)
        gsems = (semg, semg2)
        osems = (semo, semo2)

        def fetch(j, p):
            off = base + j * C
            pltpu.sync_copy(row_hbm.at[pl.ds(off, C)], ribufs[p])
            pltpu.sync_copy(col_hbm.at[pl.ds(off, C)], cibufs[p])
            ca = pltpu.async_copy(a_hbm.at[ribufs[p]], babufs[p], gsems[p])
            cb = pltpu.async_copy(b_hbm.at[cibufs[p]], bbbufs[p], gsems[p])
            return ca, cb

        def flush(j, p):
            off = base + j * C
            pltpu.async_copy(babufs[p], ga_hbm.at[pl.ds(off, C)], osems[p])
            pltpu.async_copy(bbbufs[p], gb_hbm.at[pl.ds(off, C)], osems[p])

        def wait_gather(p):
            pltpu.make_async_copy(a_hbm.at[ribufs[p]], babufs[p],
                                  gsems[p]).wait()
            pltpu.make_async_copy(b_hbm.at[cibufs[p]], bbbufs[p],
                                  gsems[p]).wait()

        def wait_out(p):
            pltpu.make_async_copy(babufs[p], ga_hbm.at[pl.ds(0, C)],
                                  osems[p]).wait()
            pltpu.make_async_copy(bbbufs[p], gb_hbm.at[pl.ds(0, C)],
                                  osems[p]).wait()

        fetch(0, 0)

        def sub(j, p, q):
            @pl.when(j + 1 < nfull)
            def _():
                @pl.when(j >= 1)
                def _():
                    wait_out(q)
                fetch(j + 1, q)
            wait_gather(p)
            flush(j, p)

        def pairstep(i2, _):
            sub(i2 * 2, 0, 1)
            sub(i2 * 2 + 1, 1, 0)
            return 0
        lax.fori_loop(0, nfull // 2, pairstep, 0)
        wait_out(0)
        wait_out(1)

        offr = base + nfull * C
        pltpu.sync_copy(row_hbm.at[pl.ds(offr, rem)], ri8)
        pltpu.sync_copy(col_hbm.at[pl.ds(offr, rem)], ci8)
        ca = pltpu.async_copy(a_hbm.at[ri8], ba8, semg)
        cb = pltpu.async_copy(b_hbm.at[ci8], bb8, semg)
        ca.wait()
        cb.wait()
        pltpu.sync_copy(ba8, ga_hbm.at[pl.ds(offr, rem)])
        pltpu.sync_copy(bb8, gb_hbm.at[pl.ds(offr, rem)])

    kern = pl.kernel(body, out_type=out_type, mesh=mesh, scratch_types=scratch)
    return kern(a, b, row, col)


def _dot(a, b):
    return jnp.dot(a, b, preferred_element_type=jnp.float32)


def _sage_linear(s, c, x_dst, Wl, bl, Wr, relu, Wproj=None, bproj=None):
    n, d = x_dst.shape
    R = 1000
    dproj = None if Wproj is None else Wproj.shape[1]

    def body(s_ref, c_ref, x_ref, wl_ref, b_ref, wr_ref, *rest):
        o_ref = rest[-1]
        cnt = c_ref[...][:, 0]
        mean = s_ref[...] / jnp.maximum(cnt, 1.0)[:, None]
        z = _dot(mean, wl_ref[...]) + b_ref[...] + _dot(x_ref[...], wr_ref[...])
        if relu:
            z = jnp.maximum(z, 0.0)
        if Wproj is not None:
            z = _dot(z, rest[0][...]) + rest[1][...]
        o_ref[...] = z

    in_specs = [
        pl.BlockSpec((R, d), lambda i: (i, 0)),
        pl.BlockSpec((R, 16), lambda i: (i, 0)),
        pl.BlockSpec((R, d), lambda i: (i, 0)),
        pl.BlockSpec((d, d), lambda i: (0, 0)),
        pl.BlockSpec((1, d), lambda i: (0, 0)),
        pl.BlockSpec((d, d), lambda i: (0, 0)),
    ]
    args = [s, c, x_dst, Wl, bl.reshape(1, d), Wr]
    dout = d
    if Wproj is not None:
        in_specs += [pl.BlockSpec((d, dproj), lambda i: (0, 0)),
                     pl.BlockSpec((1, dproj), lambda i: (0, 0))]
        args += [Wproj, bproj.reshape(1, dproj)]
        dout = dproj
    return pl.pallas_call(
        body,
        grid=(n // R,),
        in_specs=in_specs,
        out_specs=pl.BlockSpec((R, dout), lambda i: (i, 0)),
        out_shape=jax.ShapeDtypeStruct((n, dout), jnp.float32),
    )(*args)


def _decoder_mlp(ga, gb, W2, b2, W3, b3, w4row, b4):
    R = 640

    def body(ga_ref, gb_ref, w2_ref, b2_ref, w3_ref, b3_ref, w4_ref, b4_ref,
             o_ref):
        z = jnp.maximum(ga_ref[...] + gb_ref[...], 0.0)
        z = jnp.maximum(_dot(z, w2_ref[...]) + b2_ref[...], 0.0)
        z = jnp.maximum(_dot(z, w3_ref[...]) + b3_ref[...], 0.0)
        logit = jnp.sum(z * w4_ref[...], axis=1) + b4_ref[0]
        o_ref[...] = (jax.nn.sigmoid(logit) * 4.0 + 1.0)[:, None]

    return pl.pallas_call(
        body,
        grid=(ED // R,),
        in_specs=[
            pl.BlockSpec((R, HC), lambda i: (i, 0)),
            pl.BlockSpec((R, HC), lambda i: (i, 0)),
            pl.BlockSpec((HC, HC), lambda i: (0, 0)),
            pl.BlockSpec((1, HC), lambda i: (0, 0)),
            pl.BlockSpec((HC, HC), lambda i: (0, 0)),
            pl.BlockSpec((1, HC), lambda i: (0, 0)),
            pl.BlockSpec((1, HC), lambda i: (0, 0)),
            pl.BlockSpec(memory_space=pltpu.SMEM),
        ],
        out_specs=pl.BlockSpec((R, 1), lambda i: (i, 0)),
        out_shape=jax.ShapeDtypeStruct((ED, 1), jnp.float32),
    )(ga, gb, W2, b2.reshape(1, HC), W3, b3.reshape(1, HC), w4row,
      b4).reshape(ED)


def kernel(x_user, x_item, edge_index_ui, edge_index_iu, edge_label_index,
           Wl_ui1, bl_ui1, Wr_ui1, Wl_iu1, bl_iu1, Wr_iu1,
           Wl_ui2, bl_ui2, Wr_ui2, Wl_iu2, bl_iu2, Wr_iu2,
           W1, b1, W2, b2, W3, b3, W4, b4):
    src_ui, dst_ui = edge_index_ui[0], edge_index_ui[1]
    src_iu, dst_iu = edge_index_iu[0], edge_index_iu[1]
    row, col = edge_label_index[0], edge_label_index[1]

    Sui, Cui = _segsum(src_ui, dst_ui, x_user)
    Siu, Ciu = _segsum(src_iu, dst_iu, x_item)
    h_item = _sage_linear(Sui, Cui, x_item, Wl_ui1, bl_ui1, Wr_ui1, relu=True)
    h_user = _sage_linear(Siu, Ciu, x_user, Wl_iu1, bl_iu1, Wr_iu1, relu=True)

    S2ui, _ = _segsum(src_ui, dst_ui, h_user)
    S2iu, _ = _segsum(src_iu, dst_iu, h_item)
    a = _sage_linear(S2iu, Ciu, h_user, Wl_iu2, bl_iu2, Wr_iu2, relu=False,
                     Wproj=W1[:DM], bproj=jnp.zeros((HC,), jnp.float32))
    b = _sage_linear(S2ui, Cui, h_item, Wl_ui2, bl_ui2, Wr_ui2, relu=False,
                     Wproj=W1[DM:], bproj=b1)

    ga, gb = _gather_pair(a, b, row, col)
    out = _decoder_mlp(ga, gb, W2, b2, W3, b3, W4.reshape(1, HC), b4)
    mask = jnp.ones((edge_label_index.shape[1],), dtype=bool)
    return (out, mask)
